# baseline (speedup 1.0000x reference)
import functools

import jax
import jax.numpy as jnp
from jax import lax
from jax.experimental import pallas as pl
from jax.experimental.pallas import tpu as pltpu
from jax.experimental.pallas import tpu_sc as plsc

N = 320000
C = 128
S = 2048

NUM_CORES = 2
NUM_SUBCORES = 16
NW = NUM_CORES * NUM_SUBCORES
ROWS_PER_W = N // NW
CHUNK = 400
NCHUNK = ROWS_PER_W // CHUNK
SUB = 80
NSUB = CHUNK // SUB
WMAX = 16
ROWS_PER_TILE = S // NUM_SUBCORES

_mesh = plsc.VectorSubcoreMesh(core_axis_name="c", subcore_axis_name="s")
_sc_params = pltpu.CompilerParams(needs_layout_passes=False)


def _stage_idx2d(idx1d, idx2d):
    for j in range(NSUB):
        for t in range(SUB // 16):
            idx2d[j, pl.ds(t * 16, 16)] = idx1d[pl.ds(j * SUB + t * 16, 16)]


@functools.partial(
    pl.kernel,
    out_type=(
        jax.ShapeDtypeStruct((NUM_CORES, S, C), jnp.float32),
        jax.ShapeDtypeStruct((NW * S,), jnp.float32),
    ),
    mesh=_mesh,
    scratch_types=[
        pltpu.VMEM((CHUNK, C), jnp.float32),
        pltpu.VMEM((CHUNK, C), jnp.float32),
        pltpu.VMEM((CHUNK,), jnp.int32),
        pltpu.VMEM((CHUNK,), jnp.int32),
        pltpu.VMEM((NSUB, SUB), jnp.int32),
        pltpu.VMEM((NSUB, SUB), jnp.int32),
        pltpu.VMEM((S,), jnp.float32),
        pltpu.VMEM_SHARED((S, C), jnp.float32),
        pltpu.SemaphoreType.DMA,
        pltpu.SemaphoreType.DMA,
        pltpu.SemaphoreType.DMA,
        pltpu.SemaphoreType.DMA,
        pltpu.SemaphoreType.DMA,
        pltpu.SemaphoreType.DMA,
    ],
    compiler_params=_sc_params,
)
def _phase_a(x_hbm, b_hbm, zeros_hbm, psums_hbm, pcnt_hbm,
             xbuf0, xbuf1, i1d0, i1d1, i2d0, i2d1, cnt, sums_sh,
             xsem0, xsem1, isem0, isem1, ssem0, ssem1):
    cid = lax.axis_index("c")
    sid = lax.axis_index("s")
    wid = cid * NUM_SUBCORES + sid
    xbufs, i1ds, i2ds = [xbuf0, xbuf1], [i1d0, i1d1], [i2d0, i2d1]
    xsems, isems, ssems = [xsem0, xsem1], [isem0, isem1], [ssem0, ssem1]

    pltpu.sync_copy(zeros_hbm.at[pl.ds(sid * ROWS_PER_TILE, ROWS_PER_TILE)],
                    sums_sh.at[pl.ds(sid * ROWS_PER_TILE, ROWS_PER_TILE)])
    zeros16 = jnp.zeros((16,), jnp.float32)

    def zero_body(i, carry):
        cnt[pl.ds(i * 16, 16)] = zeros16
        return carry

    lax.fori_loop(0, S // 16, zero_body, 0)
    plsc.subcore_barrier()

    ones16 = jnp.ones((16,), jnp.float32)

    def fire_in(k, b):
        row0 = wid * ROWS_PER_W + k * CHUNK
        pltpu.async_copy(x_hbm.at[pl.ds(row0, CHUNK)], xbufs[b], xsems[b])
        pltpu.async_copy(b_hbm.at[pl.ds(row0, CHUNK)], i1ds[b], isems[b])

    def wait_in(b):
        pltpu.make_async_copy(x_hbm.at[pl.ds(0, CHUNK)], xbufs[b],
                              xsems[b]).wait()
        pltpu.make_async_copy(b_hbm.at[pl.ds(0, CHUNK)], i1ds[b],
                              isems[b]).wait()

    def drain_scatter(b):
        for j in range(NSUB):
            pltpu.make_async_copy(xbufs[b].at[pl.ds(j * SUB, SUB)],
                                  sums_sh.at[i2ds[b].at[j]], ssems[b]).wait()

    fire_in(0, 0)

    def outer(kk, carry):
        for b in range(2):
            k = kk * 2 + b

            @pl.when(k < NCHUNK)
            def _process():
                wait_in(b)

                @pl.when(k >= 1)
                def _drain_other():
                    drain_scatter(1 - b)

                @pl.when(k + 1 < NCHUNK)
                def _prefetch():
                    fire_in(k + 1, 1 - b)

                _stage_idx2d(i1ds[b], i2ds[b])
                for j in range(NSUB):
                    pltpu.async_copy(xbufs[b].at[pl.ds(j * SUB, SUB)],
                                     sums_sh.at[i2ds[b].at[j]],
                                     ssems[b], add=True)
                for j in range(NSUB):
                    for t in range(SUB // 16):
                        idx16 = i2ds[b][j, pl.ds(t * 16, 16)]
                        plsc.addupdate_scatter(cnt, [idx16], ones16)
        return carry

    lax.fori_loop(0, (NCHUNK + 1) // 2, outer, 0)
    drain_scatter((NCHUNK - 1) % 2)

    plsc.subcore_barrier()
    pltpu.sync_copy(sums_sh.at[pl.ds(sid * ROWS_PER_TILE, ROWS_PER_TILE)],
                    psums_hbm.at[cid, pl.ds(sid * ROWS_PER_TILE,
                                            ROWS_PER_TILE)])
    pltpu.sync_copy(cnt, pcnt_hbm.at[pl.ds(wid * S, S)])


@functools.partial(
    pl.kernel,
    out_type=jax.ShapeDtypeStruct((NUM_CORES, S, C), jnp.float32),
    mesh=_mesh,
    scratch_types=[
        pltpu.VMEM((CHUNK, C), jnp.float32),
        pltpu.VMEM((CHUNK, C), jnp.float32),
        pltpu.VMEM((CHUNK,), jnp.int32),
        pltpu.VMEM((CHUNK,), jnp.int32),
        pltpu.VMEM((NSUB, SUB), jnp.int32),
        pltpu.VMEM((NSUB, SUB), jnp.int32),
        pltpu.VMEM((WMAX, C), jnp.float32),
        pltpu.VMEM((WMAX, C), jnp.float32),
        pltpu.VMEM((WMAX, C), jnp.float32),
        pltpu.VMEM((1, 16), jnp.int32),
        pltpu.VMEM((1, WMAX), jnp.int32),
        pltpu.VMEM_SHARED((S, C), jnp.float32),
        pltpu.SemaphoreType.DMA,
        pltpu.SemaphoreType.DMA,
        pltpu.SemaphoreType.DMA,
        pltpu.SemaphoreType.DMA,
        pltpu.SemaphoreType.DMA,
        pltpu.SemaphoreType.DMA,
        pltpu.SemaphoreType.DMA,
    ],
    compiler_params=_sc_params,
)
def _phase_b(x_hbm, b_hbm, c_hbm, zeros_hbm, hpart_hbm,
             xbuf0, xbuf1, i1d0, i1d1, i2d0, i2d1, cbuf, acc0, acc1,
             i16, iacc, h_sh,
             xsem0, xsem1, isem0, isem1, ssem0, ssem1, gsem):
    cid = lax.axis_index("c")
    sid = lax.axis_index("s")
    wid = cid * NUM_SUBCORES + sid
    xbufs, i1ds, i2ds = [xbuf0, xbuf1], [i1d0, i1d1], [i2d0, i2d1]
    accs = [acc0, acc1]
    xsems, isems, ssems = [xsem0, xsem1], [isem0, isem1], [ssem0, ssem1]

    pltpu.sync_copy(zeros_hbm.at[pl.ds(sid * ROWS_PER_TILE, ROWS_PER_TILE)],
                    h_sh.at[pl.ds(sid * ROWS_PER_TILE, ROWS_PER_TILE)])
    plsc.subcore_barrier()

    lanes = lax.iota(jnp.int32, 16)
    perms = [jnp.bitwise_xor(lanes, jnp.int32(1 << bb)) for bb in range(4)]
    nq = C // 16

    def _gate4(xb, r0, crow_of):
        xs = [[xb[r0 + u, pl.ds(16 * q, 16)] for q in range(nq)]
              for u in range(4)]
        gates = []
        for u in range(4):
            cref, cr = crow_of(u)
            cs = [cref[cr, pl.ds(16 * q, 16)] for q in range(nq)]
            acc = xs[u][0] * cs[0]
            for q in range(1, nq):
                acc = acc + xs[u][q] * cs[q]
            for p in perms:
                acc = acc + jnp.take(acc, p)
            gates.append(1.0 / (1.0 + jnp.exp(-acc)))
        for u in range(4):
            for q in range(nq):
                xb[r0 + u, pl.ds(16 * q, 16)] = xs[u][q] * gates[u]

    def fire_in(k, b):
        row0 = wid * ROWS_PER_W + k * CHUNK
        pltpu.async_copy(x_hbm.at[pl.ds(row0, CHUNK)], xbufs[b], xsems[b])
        pltpu.async_copy(b_hbm.at[pl.ds(row0, CHUNK)], i1ds[b], isems[b])

    def wait_in(b):
        pltpu.make_async_copy(x_hbm.at[pl.ds(0, CHUNK)], xbufs[b],
                              xsems[b]).wait()
        pltpu.make_async_copy(b_hbm.at[pl.ds(0, CHUNK)], i1ds[b],
                              isems[b]).wait()

    def drain_scatter(b):
        pltpu.make_async_copy(accs[b], h_sh.at[iacc.at[0]], ssems[b]).wait()

    fire_in(0, 0)

    zeros16 = jnp.zeros((16,), jnp.float32)

    def outer(kk, carry):
        for b in range(2):
            k = kk * 2 + b

            @pl.when(k < NCHUNK)
            def _process():
                wait_in(b)

                @pl.when(k >= 1)
                def _drain_other():
                    drain_scatter(1 - b)

                @pl.when(k + 1 < NCHUNK)
                def _prefetch():
                    fire_in(k + 1, 1 - b)

                _stage_idx2d(i1ds[b], i2ds[b])
                lo = i1ds[b][pl.ds(0, 16)][0]
                hi = i1ds[b][pl.ds(CHUNK - 16, 16)][15]
                base = pl.multiple_of(
                    jnp.minimum((lo // 8) * 8, jnp.int32(S - WMAX)), 8)
                for w in range(WMAX):
                    for q in range(nq):
                        accs[b][w, pl.ds(16 * q, 16)] = zeros16
                iacc[0, pl.ds(0, WMAX)] = base + lanes

                def fast_path():
                    pltpu.async_copy(c_hbm.at[pl.ds(base, WMAX)], cbuf,
                                    gsem).wait()

                    @plsc.parallel_loop(0, CHUNK, step=16, unroll=1)
                    def row_body(r0):
                        bv = i1ds[b][pl.ds(r0, 16)] - base
                        for g in range(4):
                            _gate4(xbufs[b], r0 + 4 * g,
                                   lambda u: (cbuf, bv[4 * g + u]))

                    def agroup(t, acarry):
                        bv = i1ds[b][pl.ds(t * 16, 16)] - base
                        u0 = bv[0]
                        u15 = bv[15]

                        def uniform():
                            for q in range(nq):
                                s = xbufs[b][t * 16, pl.ds(16 * q, 16)]
                                for u in range(1, 16):
                                    s = s + xbufs[b][t * 16 + u,
                                                     pl.ds(16 * q, 16)]
                                accs[b][u0, pl.ds(16 * q, 16)] = (
                                    accs[b][u0, pl.ds(16 * q, 16)] + s)

                        def mixed():
                            for u in range(16):
                                cr = bv[u]
                                for q in range(nq):
                                    accs[b][cr, pl.ds(16 * q, 16)] = (
                                        accs[b][cr, pl.ds(16 * q, 16)]
                                        + xbufs[b][t * 16 + u,
                                                   pl.ds(16 * q, 16)])

                        lax.cond(u0 == u15, uniform, mixed)
                        return acarry

                    lax.fori_loop(0, CHUNK // 16, agroup, 0)

                def slow_path():
                    def sgroup(t, scarry):
                        i16[0, pl.ds(0, 16)] = i1ds[b][pl.ds(t * 16, 16)]
                        pltpu.async_copy(c_hbm.at[i16.at[0]],
                                        cbuf.at[pl.ds(0, 16)], gsem).wait()

                        @plsc.parallel_loop(0, 16, step=4, unroll=2)
                        def row_body(r0):
                            _gate4(xbufs[b], t * 16 + r0,
                                   lambda u: (cbuf, r0 + u))

                        pltpu.async_copy(
                            xbufs[b].at[pl.ds(t * 16, 16)],
                            h_sh.at[i16.at[0]], gsem, add=True).wait()
                        return scarry

                    lax.fori_loop(0, CHUNK // 16, sgroup, 0)

                lax.cond(hi - base < WMAX, fast_path, slow_path)

                pltpu.async_copy(accs[b], h_sh.at[iacc.at[0]],
                                 ssems[b], add=True)
        return carry

    lax.fori_loop(0, (NCHUNK + 1) // 2, outer, 0)
    drain_scatter((NCHUNK - 1) % 2)

    plsc.subcore_barrier()
    pltpu.sync_copy(h_sh.at[pl.ds(sid * ROWS_PER_TILE, ROWS_PER_TILE)],
                    hpart_hbm.at[cid, pl.ds(sid * ROWS_PER_TILE,
                                            ROWS_PER_TILE)])


def _mid_body(ps_ref, pc_ref, w_ref, c_ref):
    sums = ps_ref[0] + ps_ref[1]
    counts = jnp.sum(pc_ref[...].reshape(NW, S), axis=0)
    mean = sums / jnp.maximum(counts, 1.0)[:, None]
    c_ref[...] = jnp.tanh(
        jnp.dot(mean, w_ref[...], preferred_element_type=jnp.float32))


def _add_body(hp_ref, out_ref):
    out_ref[...] = hp_ref[0] + hp_ref[1]


def kernel(x, batch, weight_c):
    batch = batch.astype(jnp.int32)
    zeros = jnp.zeros((S, C), jnp.float32)

    psums, pcnt = _phase_a(x, batch, zeros)

    c = pl.pallas_call(
        _mid_body,
        out_shape=jax.ShapeDtypeStruct((S, C), jnp.float32),
    )(psums, pcnt, weight_c)

    hpart = _phase_b(x, batch, c, zeros)

    h = pl.pallas_call(
        _add_body,
        out_shape=jax.ShapeDtypeStruct((S, C), jnp.float32),
    )(hpart)
    return h


# speedup vs baseline: 1.3177x; 1.3177x over previous
import functools

import jax
import jax.numpy as jnp
from jax import lax
from jax.experimental import pallas as pl
from jax.experimental.pallas import tpu as pltpu
from jax.experimental.pallas import tpu_sc as plsc

N = 320000
C = 128
S = 2048

NUM_CORES = 2
NUM_SUBCORES = 16
NW = NUM_CORES * NUM_SUBCORES
ROWS_PER_W = N // NW
CHUNK = 400
NCHUNK = ROWS_PER_W // CHUNK
SUB = 80
NSUB = CHUNK // SUB
WMAX = 32
ROWS_PER_TILE = S // NUM_SUBCORES

_mesh = plsc.VectorSubcoreMesh(core_axis_name="c", subcore_axis_name="s")
_sc_params = pltpu.CompilerParams(needs_layout_passes=False)


def _stage_idx2d(idx1d, idx2d):
    for j in range(NSUB):
        for t in range(SUB // 16):
            idx2d[j, pl.ds(t * 16, 16)] = idx1d[pl.ds(j * SUB + t * 16, 16)]


@functools.partial(
    pl.kernel,
    out_type=(
        jax.ShapeDtypeStruct((NUM_CORES, S, C), jnp.float32),
        jax.ShapeDtypeStruct((NW * S,), jnp.float32),
    ),
    mesh=_mesh,
    scratch_types=[
        pltpu.VMEM((CHUNK, C), jnp.float32),
        pltpu.VMEM((CHUNK, C), jnp.float32),
        pltpu.VMEM((CHUNK,), jnp.int32),
        pltpu.VMEM((CHUNK,), jnp.int32),
        pltpu.VMEM((NSUB, SUB), jnp.int32),
        pltpu.VMEM((NSUB, SUB), jnp.int32),
        pltpu.VMEM((S,), jnp.float32),
        pltpu.VMEM_SHARED((S, C), jnp.float32),
        pltpu.SemaphoreType.DMA,
        pltpu.SemaphoreType.DMA,
        pltpu.SemaphoreType.DMA,
        pltpu.SemaphoreType.DMA,
        pltpu.SemaphoreType.DMA,
        pltpu.SemaphoreType.DMA,
    ],
    compiler_params=_sc_params,
)
def _phase_a(x_hbm, b_hbm, zeros_hbm, psums_hbm, pcnt_hbm,
             xbuf0, xbuf1, i1d0, i1d1, i2d0, i2d1, cnt, sums_sh,
             xsem0, xsem1, isem0, isem1, ssem0, ssem1):
    cid = lax.axis_index("c")
    sid = lax.axis_index("s")
    wid = cid * NUM_SUBCORES + sid
    xbufs, i1ds, i2ds = [xbuf0, xbuf1], [i1d0, i1d1], [i2d0, i2d1]
    xsems, isems, ssems = [xsem0, xsem1], [isem0, isem1], [ssem0, ssem1]

    pltpu.sync_copy(zeros_hbm.at[pl.ds(sid * ROWS_PER_TILE, ROWS_PER_TILE)],
                    sums_sh.at[pl.ds(sid * ROWS_PER_TILE, ROWS_PER_TILE)])
    zeros16 = jnp.zeros((16,), jnp.float32)

    def zero_body(i, carry):
        cnt[pl.ds(i * 16, 16)] = zeros16
        return carry

    lax.fori_loop(0, S // 16, zero_body, 0)
    plsc.subcore_barrier()

    ones16 = jnp.ones((16,), jnp.float32)

    def fire_in(k, b):
        row0 = wid * ROWS_PER_W + k * CHUNK
        pltpu.async_copy(x_hbm.at[pl.ds(row0, CHUNK)], xbufs[b], xsems[b])
        pltpu.async_copy(b_hbm.at[pl.ds(row0, CHUNK)], i1ds[b], isems[b])

    def wait_in(b):
        pltpu.make_async_copy(x_hbm.at[pl.ds(0, CHUNK)], xbufs[b],
                              xsems[b]).wait()
        pltpu.make_async_copy(b_hbm.at[pl.ds(0, CHUNK)], i1ds[b],
                              isems[b]).wait()

    def drain_scatter(b):
        for j in range(NSUB):
            pltpu.make_async_copy(xbufs[b].at[pl.ds(j * SUB, SUB)],
                                  sums_sh.at[i2ds[b].at[j]], ssems[b]).wait()

    fire_in(0, 0)

    def outer(kk, carry):
        for b in range(2):
            k = kk * 2 + b

            @pl.when(k < NCHUNK)
            def _process():
                wait_in(b)

                @pl.when(k >= 1)
                def _drain_other():
                    drain_scatter(1 - b)

                @pl.when(k + 1 < NCHUNK)
                def _prefetch():
                    fire_in(k + 1, 1 - b)

                _stage_idx2d(i1ds[b], i2ds[b])
                for j in range(NSUB):
                    pltpu.async_copy(xbufs[b].at[pl.ds(j * SUB, SUB)],
                                     sums_sh.at[i2ds[b].at[j]],
                                     ssems[b], add=True)
                for j in range(NSUB):
                    for t in range(SUB // 16):
                        idx16 = i2ds[b][j, pl.ds(t * 16, 16)]
                        plsc.addupdate_scatter(cnt, [idx16], ones16)
        return carry

    lax.fori_loop(0, (NCHUNK + 1) // 2, outer, 0)
    drain_scatter((NCHUNK - 1) % 2)

    plsc.subcore_barrier()
    pltpu.sync_copy(sums_sh.at[pl.ds(sid * ROWS_PER_TILE, ROWS_PER_TILE)],
                    psums_hbm.at[cid, pl.ds(sid * ROWS_PER_TILE,
                                            ROWS_PER_TILE)])
    pltpu.sync_copy(cnt, pcnt_hbm.at[pl.ds(wid * S, S)])


@functools.partial(
    pl.kernel,
    out_type=jax.ShapeDtypeStruct((NUM_CORES, S, C), jnp.float32),
    mesh=_mesh,
    scratch_types=[
        pltpu.VMEM((CHUNK, C), jnp.float32),
        pltpu.VMEM((CHUNK, C), jnp.float32),
        pltpu.VMEM((CHUNK,), jnp.int32),
        pltpu.VMEM((CHUNK,), jnp.int32),
        pltpu.VMEM((NSUB, SUB), jnp.int32),
        pltpu.VMEM((NSUB, SUB), jnp.int32),
        pltpu.VMEM((WMAX, C), jnp.float32),
        pltpu.VMEM((1, 16), jnp.int32),
        pltpu.VMEM_SHARED((S, C), jnp.float32),
        pltpu.SemaphoreType.DMA,
        pltpu.SemaphoreType.DMA,
        pltpu.SemaphoreType.DMA,
        pltpu.SemaphoreType.DMA,
        pltpu.SemaphoreType.DMA,
        pltpu.SemaphoreType.DMA,
        pltpu.SemaphoreType.DMA,
        pltpu.SemaphoreType.DMA,
    ],
    compiler_params=_sc_params,
)
def _phase_b(x_hbm, b_hbm, c_hbm, zeros_hbm, hpart_hbm,
             xbuf0, xbuf1, i1d0, i1d1, i2d0, i2d1, cbuf, i16, h_sh,
             xsem0, xsem1, isem0, isem1, ssem0, ssem1, gsem, csem):
    cid = lax.axis_index("c")
    sid = lax.axis_index("s")
    wid = cid * NUM_SUBCORES + sid
    xbufs, i1ds, i2ds = [xbuf0, xbuf1], [i1d0, i1d1], [i2d0, i2d1]
    xsems, isems, ssems = [xsem0, xsem1], [isem0, isem1], [ssem0, ssem1]

    pltpu.sync_copy(zeros_hbm.at[pl.ds(sid * ROWS_PER_TILE, ROWS_PER_TILE)],
                    h_sh.at[pl.ds(sid * ROWS_PER_TILE, ROWS_PER_TILE)])
    plsc.subcore_barrier()

    lanes = lax.iota(jnp.int32, 16)
    perms = [jnp.bitwise_xor(lanes, jnp.int32(1 << bb)) for bb in range(4)]
    nq = C // 16

    def _gate4(xb, r0, crow_of):
        xs = [[xb[r0 + u, pl.ds(16 * q, 16)] for q in range(nq)]
              for u in range(4)]
        gates = []
        for u in range(4):
            cref, cr = crow_of(u)
            cs = [cref[cr, pl.ds(16 * q, 16)] for q in range(nq)]
            acc = xs[u][0] * cs[0]
            for q in range(1, nq):
                acc = acc + xs[u][q] * cs[q]
            for p in perms:
                acc = acc + jnp.take(acc, p)
            gates.append(1.0 / (1.0 + jnp.exp(-acc)))
        for u in range(4):
            for q in range(nq):
                xb[r0 + u, pl.ds(16 * q, 16)] = xs[u][q] * gates[u]

    def fire_in(k, b):
        row0 = wid * ROWS_PER_W + k * CHUNK
        pltpu.async_copy(x_hbm.at[pl.ds(row0, CHUNK)], xbufs[b], xsems[b])
        pltpu.async_copy(b_hbm.at[pl.ds(row0, CHUNK)], i1ds[b], isems[b])

    def wait_in(b):
        pltpu.make_async_copy(x_hbm.at[pl.ds(0, CHUNK)], xbufs[b],
                              xsems[b]).wait()
        pltpu.make_async_copy(b_hbm.at[pl.ds(0, CHUNK)], i1ds[b],
                              isems[b]).wait()

    def drain_scatter(b):
        for j in range(NSUB):
            pltpu.make_async_copy(xbufs[b].at[pl.ds(j * SUB, SUB)],
                                  h_sh.at[i2ds[b].at[j]], ssems[b]).wait()

    fire_in(0, 0)

    def outer(kk, carry):
        for b in range(2):
            k = kk * 2 + b

            @pl.when(k < NCHUNK)
            def _process():
                wait_in(b)
                lo = i1ds[b][pl.ds(0, 16)][0]
                hi = i1ds[b][pl.ds(CHUNK - 16, 16)][15]
                base = pl.multiple_of(
                    jnp.minimum((lo // 8) * 8, jnp.int32(S - WMAX)), 8)
                pltpu.async_copy(c_hbm.at[pl.ds(base, WMAX)], cbuf, csem)

                @pl.when(k >= 1)
                def _drain_other():
                    drain_scatter(1 - b)

                @pl.when(k + 1 < NCHUNK)
                def _prefetch():
                    fire_in(k + 1, 1 - b)

                _stage_idx2d(i1ds[b], i2ds[b])

                def fast_path():
                    pltpu.make_async_copy(c_hbm.at[pl.ds(0, WMAX)], cbuf,
                                          csem).wait()

                    @plsc.parallel_loop(0, CHUNK, step=16, unroll=2)
                    def row_body(r0):
                        bv = i1ds[b][pl.ds(r0, 16)] - base
                        for g in range(4):
                            _gate4(xbufs[b], r0 + 4 * g,
                                   lambda u: (cbuf, bv[4 * g + u]))

                def slow_path():
                    pltpu.make_async_copy(c_hbm.at[pl.ds(0, WMAX)], cbuf,
                                          csem).wait()

                    def sgroup(t, scarry):
                        i16[0, pl.ds(0, 16)] = i1ds[b][pl.ds(t * 16, 16)]
                        pltpu.async_copy(c_hbm.at[i16.at[0]],
                                        cbuf.at[pl.ds(0, 16)], gsem).wait()

                        @plsc.parallel_loop(0, 16, step=4, unroll=2)
                        def row_body(r0):
                            _gate4(xbufs[b], t * 16 + r0,
                                   lambda u: (cbuf, r0 + u))
                        return scarry

                    lax.fori_loop(0, CHUNK // 16, sgroup, 0)

                lax.cond(hi - base < WMAX, fast_path, slow_path)

                for j in range(NSUB):
                    pltpu.async_copy(xbufs[b].at[pl.ds(j * SUB, SUB)],
                                     h_sh.at[i2ds[b].at[j]],
                                     ssems[b], add=True)
        return carry

    lax.fori_loop(0, (NCHUNK + 1) // 2, outer, 0)
    drain_scatter((NCHUNK - 1) % 2)

    plsc.subcore_barrier()
    pltpu.sync_copy(h_sh.at[pl.ds(sid * ROWS_PER_TILE, ROWS_PER_TILE)],
                    hpart_hbm.at[cid, pl.ds(sid * ROWS_PER_TILE,
                                            ROWS_PER_TILE)])


def _mid_body(ps_ref, pc_ref, w_ref, c_ref):
    sums = ps_ref[0] + ps_ref[1]
    counts = jnp.sum(pc_ref[...].reshape(NW, S), axis=0)
    mean = sums / jnp.maximum(counts, 1.0)[:, None]
    c_ref[...] = jnp.tanh(
        jnp.dot(mean, w_ref[...], preferred_element_type=jnp.float32))


def _add_body(hp_ref, out_ref):
    out_ref[...] = hp_ref[0] + hp_ref[1]


def kernel(x, batch, weight_c):
    batch = batch.astype(jnp.int32)
    zeros = jnp.zeros((S, C), jnp.float32)

    psums, pcnt = _phase_a(x, batch, zeros)

    c = pl.pallas_call(
        _mid_body,
        out_shape=jax.ShapeDtypeStruct((S, C), jnp.float32),
    )(psums, pcnt, weight_c)

    hpart = _phase_b(x, batch, c, zeros)

    h = pl.pallas_call(
        _add_body,
        out_shape=jax.ShapeDtypeStruct((S, C), jnp.float32),
    )(hpart)
    return h
